# baseline (speedup 1.0000x reference)
import jax
import jax.numpy as jnp
from jax.experimental import pallas as pl
from jax.experimental.pallas import tpu as pltpu

D = 64
K = 1024
ROWS = 4
BLK = ROWS * 1024


def _vq_block(x_ref, w_ref, out_ref, loss_ref, w2_ref, wh_ref, wsq_ref, wsh_ref):
    i = pl.program_id(0)
    g = pl.num_programs(0)

    @pl.when(i == 0)
    def _prep():
        w = w_ref[...]
        w2_ref[...] = w + w
        wh_ref[...] = 0.5 * w
        ones_d = jnp.ones((1, D), dtype=jnp.float32)
        ones_k = jnp.ones((1, K), dtype=jnp.float32)
        wsq_ref[...] = jax.lax.dot_general(
            ones_d, w * w, (((1,), (1,)), ((), ())),
            preferred_element_type=jnp.float32)
        wsh_ref[...] = 0.5 * jax.lax.dot_general(
            ones_k, w, (((1,), (0,)), ((), ())),
            preferred_element_type=jnp.float32)

    xb = x_ref[...].reshape(BLK, D)
    w2 = w2_ref[...]
    v = jax.lax.dot_general(xb, w2, (((1,), (1,)), ((), ())),
                            preferred_element_type=jnp.float32) - wsq_ref[...]
    m = jnp.max(v, axis=1, keepdims=True)
    e = jnp.exp(v)
    ones_col = jnp.ones((K, 1), dtype=jnp.float32)
    se = jax.lax.dot_general(e, ones_col, (((1,), (0,)), ((), ())),
                             preferred_element_type=jnp.float32)
    lse = jnp.log(se)
    comb = v + (v == m).astype(jnp.float32)
    ow = jax.lax.dot_general(comb, wh_ref[...], (((1,), (0,)), ((), ())),
                             preferred_element_type=jnp.float32)
    corr = jax.lax.dot_general(lse, wsh_ref[...], (((1,), (0,)), ((), ())),
                               preferred_element_type=jnp.float32)
    out_ref[...] = (ow - corr).reshape(out_ref.shape)
    ones_d = jnp.ones((D, 1), dtype=jnp.float32)
    xsq = jax.lax.dot_general(xb * xb, ones_d, (((1,), (0,)), ((), ())),
                              preferred_element_type=jnp.float32)
    part = jnp.sum(xsq - m)
    acc = jnp.where(i == 0, 0.0, loss_ref[0]) + part
    loss_ref[0] = jnp.where(i == g - 1, acc * (1.25 / (g * BLK * D)), acc)


def kernel(x, codebook):
    b, s, _ = x.shape
    n = b * s
    out, loss = pl.pallas_call(
        _vq_block,
        grid=(n // BLK,),
        in_specs=[pl.BlockSpec((ROWS, 1024, D), lambda i: (i, 0, 0)),
                  pl.BlockSpec((K, D), lambda i: (0, 0))],
        out_specs=[pl.BlockSpec((ROWS, 1024, D), lambda i: (i, 0, 0)),
                   pl.BlockSpec(memory_space=pltpu.SMEM)],
        out_shape=[jax.ShapeDtypeStruct((b, s, D), jnp.float32),
                   jax.ShapeDtypeStruct((1,), jnp.float32)],
        scratch_shapes=[pltpu.VMEM((K, D), jnp.float32),
                        pltpu.VMEM((K, D), jnp.float32),
                        pltpu.VMEM((1, K), jnp.float32),
                        pltpu.VMEM((1, D), jnp.float32)],
    )(x, codebook)
    return out, loss[0]


# speedup vs baseline: 1.0072x; 1.0072x over previous
import jax
import jax.numpy as jnp
from jax.experimental import pallas as pl
from jax.experimental.pallas import tpu as pltpu

D = 64
K = 1024
ROWS = 4
BLK = ROWS * 1024


def _vq_block(x_ref, w_ref, out_ref, loss_ref, w2_ref, wsq_ref, wsh_ref):
    i = pl.program_id(0)
    g = pl.num_programs(0)

    @pl.when(i == 0)
    def _prep():
        w = w_ref[...]
        w2_ref[...] = w + w
        ones_d = jnp.ones((1, D), dtype=jnp.float32)
        ones_k = jnp.ones((1, K), dtype=jnp.float32)
        wsq_ref[...] = jax.lax.dot_general(
            ones_d, w * w, (((1,), (1,)), ((), ())),
            preferred_element_type=jnp.float32)
        wsh_ref[...] = 0.5 * jax.lax.dot_general(
            ones_k, w, (((1,), (0,)), ((), ())),
            preferred_element_type=jnp.float32)

    xb = x_ref[...].reshape(BLK, D)
    w2 = w2_ref[...]
    v = jax.lax.dot_general(xb, w2, (((1,), (1,)), ((), ())),
                            preferred_element_type=jnp.float32) - wsq_ref[...]
    m = jnp.max(v, axis=1, keepdims=True)
    e = jnp.exp(v)
    ones_col = jnp.ones((K, 1), dtype=jnp.float32)
    se = jax.lax.dot_general(e, ones_col, (((1,), (0,)), ((), ())),
                             preferred_element_type=jnp.float32)
    lse = jnp.log(se)
    comb = v + (v == m).astype(jnp.float32)
    ow2 = jax.lax.dot_general(comb, w2, (((1,), (0,)), ((), ())),
                              preferred_element_type=jnp.float32)
    corr = jax.lax.dot_general(lse, wsh_ref[...], (((1,), (0,)), ((), ())),
                               preferred_element_type=jnp.float32)
    out_ref[...] = (0.25 * ow2 - corr).reshape(out_ref.shape)
    t = jnp.sum(xb * xb, axis=1, keepdims=True) - m
    part = jnp.sum(t)
    acc = jnp.where(i == 0, 0.0, loss_ref[0]) + part
    loss_ref[0] = jnp.where(i == g - 1, acc * (1.25 / (g * BLK * D)), acc)


def kernel(x, codebook):
    b, s, _ = x.shape
    n = b * s
    out, loss = pl.pallas_call(
        _vq_block,
        grid=(n // BLK,),
        in_specs=[pl.BlockSpec((ROWS, 1024, D), lambda i: (i, 0, 0)),
                  pl.BlockSpec((K, D), lambda i: (0, 0))],
        out_specs=[pl.BlockSpec((ROWS, 1024, D), lambda i: (i, 0, 0)),
                   pl.BlockSpec(memory_space=pltpu.SMEM)],
        out_shape=[jax.ShapeDtypeStruct((b, s, D), jnp.float32),
                   jax.ShapeDtypeStruct((1,), jnp.float32)],
        scratch_shapes=[pltpu.VMEM((K, D), jnp.float32),
                        pltpu.VMEM((1, K), jnp.float32),
                        pltpu.VMEM((1, D), jnp.float32)],
    )(x, codebook)
    return out, loss[0]
